# baseline (speedup 1.0000x reference)
import functools

import jax
import jax.numpy as jnp
from jax import lax
from jax.experimental import pallas as pl
from jax.experimental.pallas import tpu as pltpu
from jax.experimental.pallas import tpu_sc as plsc

B, T, D, V, L = 32, 1000, 512, 256, 100
S = 2 * L + 1
VK = 64
NV = 13
SPAD = 16 * NV
AB = 16
ABUF = AB + SPAD
PBIG = 1 << 30
TBP = 40
NCHUNK = T // TBP
NEGF = -1e30
LN2 = 0.6931471805599453


def _proj_kernel(hs_ref, w_ref, b_ref, p_ref):
    hs = hs_ref[...].reshape(B * TBP, D)
    logits = jnp.dot(hs, w_ref[...], preferred_element_type=jnp.float32)
    logits = logits + b_ref[...]
    m = jnp.max(logits, axis=-1, keepdims=True)
    lse = m + jnp.log(jnp.sum(jnp.exp(logits - m), axis=-1, keepdims=True))
    p_ref[...] = jnp.exp(logits[:, :VK] - lse).reshape(B, TBP, VK)


def _sc_body(p_hbm, ext_hbm, skip_hbm, hl_hbm,
             mfin_hbm, pfin_hbm,
             pv, extv, skipv, hlv, mbuf, pbuf):
    b = lax.axis_index("s") * 2 + lax.axis_index("c")
    pltpu.sync_copy(p_hbm.at[b], pv)
    pltpu.sync_copy(ext_hbm.at[b], extv)
    pltpu.sync_copy(skip_hbm.at[b], skipv)
    pltpu.sync_copy(hl_hbm.at[b], hlv)

    iota = lax.iota(jnp.int32, 16)
    zero = jnp.zeros((16,), jnp.float32)
    negbig = jnp.full((16,), -PBIG, jnp.int32)

    def pow2(k):
        return plsc.bitcast((k + 127) * (1 << 23), jnp.float32)

    def expfield(v):
        return lax.shift_right_logical(plsc.bitcast(v, jnp.int32), 23)

    for k in range(AB // 16):
        mbuf[pl.ds(16 * k, 16)] = zero
        pbuf[pl.ds(16 * k, 16)] = negbig
    for k in range(NV):
        mbuf[pl.ds(AB + 16 * k, 16)] = zero
        pbuf[pl.ds(AB + 16 * k, 16)] = negbig

    e0 = extv[pl.ds(0, 16)]
    g0 = plsc.load_gather(pv, [e0])
    m0 = jnp.where(iota < 2, g0, zero)
    eb0 = expfield(m0)
    mbuf[pl.ds(AB, 16)] = m0 * pow2(127 - eb0)
    pbuf[pl.ds(AB, 16)] = jnp.where(iota < 2, eb0 - 127, negbig)

    def step(t, carry):
        padmask = lax.iota(jnp.int32, 16) < (S - 16 * (NV - 1))
        hl = hlv[...]
        ts = jnp.full((16,), t, dtype=jnp.int32)
        run = ts < hl
        toff = jnp.full((16,), t * VK, dtype=jnp.int32)
        c126 = jnp.full((16,), -126, jnp.int32)
        zf = jnp.zeros((16,), jnp.float32)
        nb = jnp.full((16,), -PBIG, jnp.int32)
        news = []
        for k in range(NV):
            base = AB + 16 * k
            m_s = mbuf[pl.ds(base, 16)]
            m_1 = mbuf[pl.ds(base - 1, 16)]
            m_2 = mbuf[pl.ds(base - 2, 16)]
            p_s = pbuf[pl.ds(base, 16)]
            p_1 = pbuf[pl.ds(base - 1, 16)]
            p_2 = pbuf[pl.ds(base - 2, 16)]
            sk = skipv[pl.ds(16 * k, 16)] > 0.5
            m_2 = jnp.where(sk, m_2, zf)
            p_2 = jnp.where(sk, p_2, nb)
            ek = extv[pl.ds(16 * k, 16)]
            g = plsc.load_gather(pv, [toff + ek])
            pm = jnp.maximum(jnp.maximum(p_s, p_1), p_2)
            f_s = pow2(jnp.maximum(p_s - pm, c126))
            f_1 = pow2(jnp.maximum(p_1 - pm, c126))
            f_2 = pow2(jnp.maximum(p_2 - pm, c126))
            v = (m_s * f_s + m_1 * f_1 + m_2 * f_2) * g
            if k == NV - 1:
                v = jnp.where(padmask, v, zf)
            news.append((v, pm, m_s, p_s))
        for k in range(NV):
            v, pm, m_s, p_s = news[k]
            eb = expfield(v)
            mn = v * pow2(127 - eb)
            pn = pm + (eb - 127)
            mbuf[pl.ds(AB + 16 * k, 16)] = jnp.where(run, mn, m_s)
            pbuf[pl.ds(AB + 16 * k, 16)] = jnp.where(run, pn, p_s)
        return carry

    lax.fori_loop(1, T, step, 0, unroll=False)

    pltpu.sync_copy(mbuf, mfin_hbm.at[b])
    pltpu.sync_copy(pbuf, pfin_hbm.at[b])


def _finish_kernel(m_ref, p_ref, sel_ref, sel2_ref, out_ref):
    m = m_ref[...]
    pe = p_ref[...].astype(jnp.float32)
    la = jnp.where(m > 0, jnp.log(m) + pe * LN2, NEGF)
    a_end = jnp.sum(jnp.where(sel_ref[...] > 0, la, 0.0),
                    axis=1, keepdims=True)
    a_end2 = jnp.sum(jnp.where(sel2_ref[...] > 0, la, 0.0),
                     axis=1, keepdims=True)
    nll = -jnp.logaddexp(a_end, a_end2)
    out_ref[...] = jnp.sum(nll, axis=(0, 1), keepdims=True) / B


@jax.jit
def kernel(hs_pad, hlens, ys_pad, ys_lens, W, b):
    ext = jnp.zeros((B, S), dtype=jnp.int32).at[:, 1::2].set(
        ys_pad.astype(jnp.int32))
    same = jnp.concatenate(
        [jnp.ones((B, 2), dtype=bool), ext[:, 2:] == ext[:, :-2]], axis=1)
    blank_pos = (jnp.arange(S) % 2 == 0)[None, :]
    skip_ok = jnp.logical_and(jnp.logical_not(blank_pos),
                              jnp.logical_not(same))
    ext_pad = jnp.zeros((B, SPAD), jnp.int32).at[:, :S].set(ext)
    skip_pad = jnp.zeros((B, SPAD), jnp.float32).at[:, :S].set(
        skip_ok.astype(jnp.float32))
    hl16 = jnp.broadcast_to(hlens.astype(jnp.int32)[:, None], (B, 16))

    s_last = 2 * ys_lens
    cols = jnp.arange(ABUF, dtype=jnp.int32)[None, :] - AB
    sel = (cols == s_last[:, None]).astype(jnp.float32)
    sel2 = (cols == jnp.maximum(s_last - 1, 0)[:, None]).astype(jnp.float32)
    b2 = b.reshape(1, V)

    probs = pl.pallas_call(
        _proj_kernel,
        grid=(NCHUNK,),
        in_specs=[
            pl.BlockSpec((B, TBP, D), lambda i: (0, i, 0)),
            pl.BlockSpec((D, V), lambda i: (0, 0)),
            pl.BlockSpec((1, V), lambda i: (0, 0)),
        ],
        out_specs=pl.BlockSpec((B, TBP, VK), lambda i: (0, i, 0)),
        out_shape=jax.ShapeDtypeStruct((B, T, VK), jnp.float32),
        compiler_params=pltpu.CompilerParams(
            dimension_semantics=("arbitrary",),
        ),
    )(hs_pad, W, b2)

    mesh = plsc.VectorSubcoreMesh(core_axis_name="c", subcore_axis_name="s")
    sc = pl.kernel(
        _sc_body,
        mesh=mesh,
        compiler_params=pltpu.CompilerParams(needs_layout_passes=False),
        out_type=[
            jax.ShapeDtypeStruct((B, ABUF), jnp.float32),
            jax.ShapeDtypeStruct((B, ABUF), jnp.int32),
        ],
        scratch_types=[
            pltpu.VMEM((T * VK,), jnp.float32),
            pltpu.VMEM((SPAD,), jnp.int32),
            pltpu.VMEM((SPAD,), jnp.float32),
            pltpu.VMEM((16,), jnp.int32),
            pltpu.VMEM((ABUF,), jnp.float32),
            pltpu.VMEM((ABUF,), jnp.int32),
        ],
    )
    mfin, pfin = sc(probs.reshape(B, T * VK), ext_pad, skip_pad, hl16)

    out = pl.pallas_call(
        _finish_kernel,
        in_specs=[
            pl.BlockSpec((B, ABUF), lambda: (0, 0)),
            pl.BlockSpec((B, ABUF), lambda: (0, 0)),
            pl.BlockSpec((B, ABUF), lambda: (0, 0)),
            pl.BlockSpec((B, ABUF), lambda: (0, 0)),
        ],
        out_specs=pl.BlockSpec((1, 1), lambda: (0, 0)),
        out_shape=jax.ShapeDtypeStruct((1, 1), jnp.float32),
    )(mfin, pfin, sel, sel2)
    return out[0, 0]


# speedup vs baseline: 1.0568x; 1.0568x over previous
import functools

import jax
import jax.numpy as jnp
from jax import lax
from jax.experimental import pallas as pl
from jax.experimental.pallas import tpu as pltpu
from jax.experimental.pallas import tpu_sc as plsc

B, T, D, V, L = 32, 1000, 512, 256, 100
S = 2 * L + 1
VK = 64
NV = 13
SPAD = 16 * NV
AB = 16
ABUF = AB + SPAD
PBIG = 1 << 30
TBP = 40
NCHUNK = T // TBP
NEGF = -1e30
LN2 = 0.6931471805599453


def _proj_kernel(hs_ref, w_ref, b_ref, p_ref):
    hs = hs_ref[...].reshape(B * TBP, D)
    logits = jnp.dot(hs, w_ref[...], preferred_element_type=jnp.float32)
    logits = logits + b_ref[...]
    m = jnp.max(logits, axis=-1, keepdims=True)
    lse = m + jnp.log(jnp.sum(jnp.exp(logits - m), axis=-1, keepdims=True))
    p_ref[...] = jnp.exp(logits[:, :VK] - lse).reshape(B, TBP, VK)


def _sc_body(p_hbm, ext_hbm, skip_hbm, hl_hbm,
             mfin_hbm, pfin_hbm,
             pv, extv, skipv, hlv, mbuf, pbuf):
    b = lax.axis_index("s") * 2 + lax.axis_index("c")
    pltpu.sync_copy(p_hbm.at[b], pv)
    pltpu.sync_copy(ext_hbm.at[b], extv)
    pltpu.sync_copy(skip_hbm.at[b], skipv)
    pltpu.sync_copy(hl_hbm.at[b], hlv)

    iota = lax.iota(jnp.int32, 16)
    zero = jnp.zeros((16,), jnp.float32)
    negbig = jnp.full((16,), -PBIG, jnp.int32)

    def pow2(k):
        return plsc.bitcast((k + 127) * (1 << 23), jnp.float32)

    def expfield(v):
        return lax.shift_right_logical(plsc.bitcast(v, jnp.int32), 23)

    for k in range(AB // 16):
        mbuf[pl.ds(16 * k, 16)] = zero
        pbuf[pl.ds(16 * k, 16)] = negbig
    for k in range(NV):
        mbuf[pl.ds(AB + 16 * k, 16)] = zero
        pbuf[pl.ds(AB + 16 * k, 16)] = negbig

    e0 = extv[pl.ds(0, 16)]
    g0 = plsc.load_gather(pv, [e0])
    m0 = jnp.where(iota < 2, g0, zero)
    eb0 = expfield(m0)
    mbuf[pl.ds(AB, 16)] = m0 * pow2(127 - eb0)
    pbuf[pl.ds(AB, 16)] = jnp.where(iota < 2, eb0 - 127, negbig)

    def step(t, carry):
        padmask = lax.iota(jnp.int32, 16) < (S - 16 * (NV - 1))
        toff = jnp.full((16,), t * VK, dtype=jnp.int32)
        c126 = jnp.full((16,), -126, jnp.int32)
        zf = jnp.zeros((16,), jnp.float32)
        nb = jnp.full((16,), -PBIG, jnp.int32)
        news = []
        for k in range(NV):
            base = AB + 16 * k
            m_s = mbuf[pl.ds(base, 16)]
            m_1 = mbuf[pl.ds(base - 1, 16)]
            m_2 = mbuf[pl.ds(base - 2, 16)]
            p_s = pbuf[pl.ds(base, 16)]
            p_1 = pbuf[pl.ds(base - 1, 16)]
            p_2 = pbuf[pl.ds(base - 2, 16)]
            sk = skipv[pl.ds(16 * k, 16)] > 0.5
            m_2 = jnp.where(sk, m_2, zf)
            p_2 = jnp.where(sk, p_2, nb)
            ek = extv[pl.ds(16 * k, 16)]
            g = plsc.load_gather(pv, [toff + ek])
            pm = jnp.maximum(jnp.maximum(p_s, p_1), p_2)
            f_s = pow2(jnp.maximum(p_s - pm, c126))
            f_1 = pow2(jnp.maximum(p_1 - pm, c126))
            f_2 = pow2(jnp.maximum(p_2 - pm, c126))
            v = (m_s * f_s + m_1 * f_1 + m_2 * f_2) * g
            if k == NV - 1:
                v = jnp.where(padmask, v, zf)
            news.append((v, pm))
        for k in range(NV):
            v, pm = news[k]
            eb = expfield(v)
            mbuf[pl.ds(AB + 16 * k, 16)] = v * pow2(127 - eb)
            pbuf[pl.ds(AB + 16 * k, 16)] = pm + (eb - 127)
        return carry

    hls = jnp.max(hlv[...])
    lax.fori_loop(1, hls, step, 0, unroll=False)

    pltpu.sync_copy(mbuf, mfin_hbm.at[b])
    pltpu.sync_copy(pbuf, pfin_hbm.at[b])


def _finish_kernel(m_ref, p_ref, sel_ref, sel2_ref, out_ref):
    m = m_ref[...]
    pe = p_ref[...].astype(jnp.float32)
    la = jnp.where(m > 0, jnp.log(m) + pe * LN2, NEGF)
    a_end = jnp.sum(jnp.where(sel_ref[...] > 0, la, 0.0),
                    axis=1, keepdims=True)
    a_end2 = jnp.sum(jnp.where(sel2_ref[...] > 0, la, 0.0),
                     axis=1, keepdims=True)
    nll = -jnp.logaddexp(a_end, a_end2)
    out_ref[...] = jnp.sum(nll, axis=(0, 1), keepdims=True) / B


@jax.jit
def kernel(hs_pad, hlens, ys_pad, ys_lens, W, b):
    ext = jnp.zeros((B, S), dtype=jnp.int32).at[:, 1::2].set(
        ys_pad.astype(jnp.int32))
    same = jnp.concatenate(
        [jnp.ones((B, 2), dtype=bool), ext[:, 2:] == ext[:, :-2]], axis=1)
    blank_pos = (jnp.arange(S) % 2 == 0)[None, :]
    skip_ok = jnp.logical_and(jnp.logical_not(blank_pos),
                              jnp.logical_not(same))
    ext_pad = jnp.zeros((B, SPAD), jnp.int32).at[:, :S].set(ext)
    skip_pad = jnp.zeros((B, SPAD), jnp.float32).at[:, :S].set(
        skip_ok.astype(jnp.float32))
    hl16 = jnp.broadcast_to(hlens.astype(jnp.int32)[:, None], (B, 16))

    s_last = 2 * ys_lens
    cols = jnp.arange(ABUF, dtype=jnp.int32)[None, :] - AB
    sel = (cols == s_last[:, None]).astype(jnp.float32)
    sel2 = (cols == jnp.maximum(s_last - 1, 0)[:, None]).astype(jnp.float32)
    b2 = b.reshape(1, V)

    probs = pl.pallas_call(
        _proj_kernel,
        grid=(NCHUNK,),
        in_specs=[
            pl.BlockSpec((B, TBP, D), lambda i: (0, i, 0)),
            pl.BlockSpec((D, V), lambda i: (0, 0)),
            pl.BlockSpec((1, V), lambda i: (0, 0)),
        ],
        out_specs=pl.BlockSpec((B, TBP, VK), lambda i: (0, i, 0)),
        out_shape=jax.ShapeDtypeStruct((B, T, VK), jnp.float32),
        compiler_params=pltpu.CompilerParams(
            dimension_semantics=("arbitrary",),
        ),
    )(hs_pad, W, b2)

    mesh = plsc.VectorSubcoreMesh(core_axis_name="c", subcore_axis_name="s")
    sc = pl.kernel(
        _sc_body,
        mesh=mesh,
        compiler_params=pltpu.CompilerParams(needs_layout_passes=False),
        out_type=[
            jax.ShapeDtypeStruct((B, ABUF), jnp.float32),
            jax.ShapeDtypeStruct((B, ABUF), jnp.int32),
        ],
        scratch_types=[
            pltpu.VMEM((T * VK,), jnp.float32),
            pltpu.VMEM((SPAD,), jnp.int32),
            pltpu.VMEM((SPAD,), jnp.float32),
            pltpu.VMEM((16,), jnp.int32),
            pltpu.VMEM((ABUF,), jnp.float32),
            pltpu.VMEM((ABUF,), jnp.int32),
        ],
    )
    mfin, pfin = sc(probs.reshape(B, T * VK), ext_pad, skip_pad, hl16)

    out = pl.pallas_call(
        _finish_kernel,
        in_specs=[
            pl.BlockSpec((B, ABUF), lambda: (0, 0)),
            pl.BlockSpec((B, ABUF), lambda: (0, 0)),
            pl.BlockSpec((B, ABUF), lambda: (0, 0)),
            pl.BlockSpec((B, ABUF), lambda: (0, 0)),
        ],
        out_specs=pl.BlockSpec((1, 1), lambda: (0, 0)),
        out_shape=jax.ShapeDtypeStruct((1, 1), jnp.float32),
    )(mfin, pfin, sel, sel2)
    return out[0, 0]


# speedup vs baseline: 1.0984x; 1.0394x over previous
import functools

import jax
import jax.numpy as jnp
from jax import lax
from jax.experimental import pallas as pl
from jax.experimental.pallas import tpu as pltpu
from jax.experimental.pallas import tpu_sc as plsc

B, T, D, V, L = 32, 1000, 512, 256, 100
S = 2 * L + 1
VK = 64
NV = 13
SPAD = 16 * NV
AB = 16
ABUF = AB + SPAD
PBIG = 1 << 30
TBP = 40
NCHUNK = T // TBP
NEGF = -1e30
LN2 = 0.6931471805599453


def _proj_kernel(hs_ref, w_ref, b_ref, p_ref):
    hs = hs_ref[...].reshape(B * TBP, D)
    logits = jnp.dot(hs, w_ref[...], preferred_element_type=jnp.float32)
    logits = logits + b_ref[...]
    m = jnp.max(logits, axis=-1, keepdims=True)
    lse = m + jnp.log(jnp.sum(jnp.exp(logits - m), axis=-1, keepdims=True))
    p_ref[...] = jnp.exp(logits[:, :VK] - lse).reshape(B, TBP, VK)


def _sc_body(p_hbm, ext_hbm, skip_hbm, hl_hbm,
             mfin_hbm, pfin_hbm,
             pv, extv, skipv, hlv, mbuf, pbuf):
    b = lax.axis_index("s") * 2 + lax.axis_index("c")
    pltpu.sync_copy(p_hbm.at[b], pv)
    pltpu.sync_copy(ext_hbm.at[b], extv)
    pltpu.sync_copy(skip_hbm.at[b], skipv)
    pltpu.sync_copy(hl_hbm.at[b], hlv)

    iota = lax.iota(jnp.int32, 16)
    zero = jnp.zeros((16,), jnp.float32)
    negbig = jnp.full((16,), -PBIG, jnp.int32)

    def pow2(k):
        return plsc.bitcast((k + 127) * (1 << 23), jnp.float32)

    def expfield(v):
        return lax.shift_right_logical(plsc.bitcast(v, jnp.int32), 23)

    for k in range(AB // 16):
        mbuf[pl.ds(16 * k, 16)] = zero
        pbuf[pl.ds(16 * k, 16)] = negbig
    for k in range(NV):
        mbuf[pl.ds(AB + 16 * k, 16)] = zero
        pbuf[pl.ds(AB + 16 * k, 16)] = negbig

    e0 = extv[pl.ds(0, 16)]
    g0 = plsc.load_gather(pv, [e0])
    m0 = jnp.where(iota < 2, g0, zero)
    eb0 = expfield(m0)
    mbuf[pl.ds(AB, 16)] = m0 * pow2(127 - eb0)
    pbuf[pl.ds(AB, 16)] = jnp.where(iota < 2, eb0 - 127, negbig)

    eks = [extv[pl.ds(16 * k, 16)] for k in range(NV)]
    sks = [skipv[pl.ds(16 * k, 16)] > 0.5 for k in range(NV)]
    padmask = iota < (S - 16 * (NV - 1))
    c126 = jnp.full((16,), -126, jnp.int32)
    zf = zero
    nb = negbig
    vkvec = jnp.full((16,), VK, jnp.int32)

    def step(t, carry):
        toff = jnp.full((16,), t, dtype=jnp.int32) * vkvec
        news = []
        for k in range(NV):
            base = AB + 16 * k
            m_s = mbuf[pl.ds(base, 16)]
            m_1 = mbuf[pl.ds(base - 1, 16)]
            m_2 = mbuf[pl.ds(base - 2, 16)]
            p_s = pbuf[pl.ds(base, 16)]
            p_1 = pbuf[pl.ds(base - 1, 16)]
            p_2 = pbuf[pl.ds(base - 2, 16)]
            p_2 = jnp.where(sks[k], p_2, nb)
            g = plsc.load_gather(pv, [toff + eks[k]])
            pm = jnp.maximum(jnp.maximum(p_s, p_1), p_2)
            f_s = pow2(jnp.maximum(p_s - pm, c126))
            f_1 = pow2(jnp.maximum(p_1 - pm, c126))
            f_2 = pow2(jnp.maximum(p_2 - pm, c126))
            v = (m_s * f_s + m_1 * f_1 + m_2 * f_2) * g
            if k == NV - 1:
                v = jnp.where(padmask, v, zf)
            news.append((v, pm))
        for k in range(NV):
            v, pm = news[k]
            eb = expfield(v)
            mbuf[pl.ds(AB + 16 * k, 16)] = v * pow2(127 - eb)
            pbuf[pl.ds(AB + 16 * k, 16)] = pm + (eb - 127)
        return carry

    hls = jnp.max(hlv[...])
    lax.fori_loop(1, hls, step, 0, unroll=False)

    pltpu.sync_copy(mbuf, mfin_hbm.at[b])
    pltpu.sync_copy(pbuf, pfin_hbm.at[b])


def _finish_kernel(m_ref, p_ref, sel_ref, sel2_ref, out_ref):
    m = m_ref[...]
    pe = p_ref[...].astype(jnp.float32)
    la = jnp.where(m > 0, jnp.log(m) + pe * LN2, NEGF)
    a_end = jnp.sum(jnp.where(sel_ref[...] > 0, la, 0.0),
                    axis=1, keepdims=True)
    a_end2 = jnp.sum(jnp.where(sel2_ref[...] > 0, la, 0.0),
                     axis=1, keepdims=True)
    nll = -jnp.logaddexp(a_end, a_end2)
    out_ref[...] = jnp.sum(nll, axis=(0, 1), keepdims=True) / B


@jax.jit
def kernel(hs_pad, hlens, ys_pad, ys_lens, W, b):
    ext = jnp.zeros((B, S), dtype=jnp.int32).at[:, 1::2].set(
        ys_pad.astype(jnp.int32))
    same = jnp.concatenate(
        [jnp.ones((B, 2), dtype=bool), ext[:, 2:] == ext[:, :-2]], axis=1)
    blank_pos = (jnp.arange(S) % 2 == 0)[None, :]
    skip_ok = jnp.logical_and(jnp.logical_not(blank_pos),
                              jnp.logical_not(same))
    ext_pad = jnp.zeros((B, SPAD), jnp.int32).at[:, :S].set(ext)
    skip_pad = jnp.zeros((B, SPAD), jnp.float32).at[:, :S].set(
        skip_ok.astype(jnp.float32))
    hl16 = jnp.broadcast_to(hlens.astype(jnp.int32)[:, None], (B, 16))

    s_last = 2 * ys_lens
    cols = jnp.arange(ABUF, dtype=jnp.int32)[None, :] - AB
    sel = (cols == s_last[:, None]).astype(jnp.float32)
    sel2 = (cols == jnp.maximum(s_last - 1, 0)[:, None]).astype(jnp.float32)
    b2 = b.reshape(1, V)

    probs = pl.pallas_call(
        _proj_kernel,
        grid=(NCHUNK,),
        in_specs=[
            pl.BlockSpec((B, TBP, D), lambda i: (0, i, 0)),
            pl.BlockSpec((D, V), lambda i: (0, 0)),
            pl.BlockSpec((1, V), lambda i: (0, 0)),
        ],
        out_specs=pl.BlockSpec((B, TBP, VK), lambda i: (0, i, 0)),
        out_shape=jax.ShapeDtypeStruct((B, T, VK), jnp.float32),
        compiler_params=pltpu.CompilerParams(
            dimension_semantics=("parallel",),
        ),
    )(hs_pad, W, b2)

    mesh = plsc.VectorSubcoreMesh(core_axis_name="c", subcore_axis_name="s")
    sc = pl.kernel(
        _sc_body,
        mesh=mesh,
        compiler_params=pltpu.CompilerParams(needs_layout_passes=False),
        out_type=[
            jax.ShapeDtypeStruct((B, ABUF), jnp.float32),
            jax.ShapeDtypeStruct((B, ABUF), jnp.int32),
        ],
        scratch_types=[
            pltpu.VMEM((T * VK,), jnp.float32),
            pltpu.VMEM((SPAD,), jnp.int32),
            pltpu.VMEM((SPAD,), jnp.float32),
            pltpu.VMEM((16,), jnp.int32),
            pltpu.VMEM((ABUF,), jnp.float32),
            pltpu.VMEM((ABUF,), jnp.int32),
        ],
    )
    mfin, pfin = sc(probs.reshape(B, T * VK), ext_pad, skip_pad, hl16)

    out = pl.pallas_call(
        _finish_kernel,
        in_specs=[
            pl.BlockSpec((B, ABUF), lambda: (0, 0)),
            pl.BlockSpec((B, ABUF), lambda: (0, 0)),
            pl.BlockSpec((B, ABUF), lambda: (0, 0)),
            pl.BlockSpec((B, ABUF), lambda: (0, 0)),
        ],
        out_specs=pl.BlockSpec((1, 1), lambda: (0, 0)),
        out_shape=jax.ShapeDtypeStruct((1, 1), jnp.float32),
    )(mfin, pfin, sel, sel2)
    return out[0, 0]


# speedup vs baseline: 1.1821x; 1.0762x over previous
import functools

import jax
import jax.numpy as jnp
from jax import lax
from jax.experimental import pallas as pl
from jax.experimental.pallas import tpu as pltpu
from jax.experimental.pallas import tpu_sc as plsc

B, T, D, V, L = 32, 1000, 512, 256, 100
S = 2 * L + 1
VK = 64
NV = 13
SPAD = 16 * NV
AB = 16
ABUF = AB + SPAD
PBIG = 1 << 30
TBP = 40
NCHUNK = T // TBP
NEGF = -1e30
LN2 = 0.6931471805599453


def _proj_kernel(hs_ref, w_ref, b_ref, p_ref):
    hs = hs_ref[...].reshape(B * TBP, D)
    logits = jnp.dot(hs, w_ref[...], preferred_element_type=jnp.float32)
    logits = logits + b_ref[...]
    m = jnp.max(logits, axis=-1, keepdims=True)
    lse = m + jnp.log(jnp.sum(jnp.exp(logits - m), axis=-1, keepdims=True))
    p_ref[...] = jnp.exp(logits[:, :VK] - lse).reshape(B, TBP, VK)


def _sc_body(p_hbm, ext_hbm, skip_hbm, hl_hbm,
             mfin_hbm, pfin_hbm,
             pv, extv, skipv, hlv, mbuf, pbuf):
    b = lax.axis_index("s") * 2 + lax.axis_index("c")
    pltpu.sync_copy(p_hbm.at[b], pv)
    pltpu.sync_copy(ext_hbm.at[b], extv)
    pltpu.sync_copy(skip_hbm.at[b], skipv)
    pltpu.sync_copy(hl_hbm.at[b], hlv)

    iota = lax.iota(jnp.int32, 16)
    zero = jnp.zeros((16,), jnp.float32)
    negbig = jnp.full((16,), -PBIG, jnp.int32)

    def pow2(k):
        return plsc.bitcast((k + 127) * (1 << 23), jnp.float32)

    def expfield(v):
        return lax.shift_right_logical(plsc.bitcast(v, jnp.int32), 23)

    for k in range(AB // 16):
        mbuf[pl.ds(16 * k, 16)] = zero
        pbuf[pl.ds(16 * k, 16)] = negbig
    for k in range(NV):
        mbuf[pl.ds(AB + 16 * k, 16)] = zero
        pbuf[pl.ds(AB + 16 * k, 16)] = negbig

    e0 = extv[pl.ds(0, 16)]
    g0 = plsc.load_gather(pv, [e0])
    m0 = jnp.where(iota < 2, g0, zero)
    eb0 = expfield(m0)
    mbuf[pl.ds(AB, 16)] = m0 * pow2(127 - eb0)
    pbuf[pl.ds(AB, 16)] = jnp.where(iota < 2, eb0 - 127, negbig)

    eks = [extv[pl.ds(16 * k, 16)] for k in range(NV)]
    sks = [skipv[pl.ds(16 * k, 16)] > 0.5 for k in range(NV)]
    padmask = iota < (S - 16 * (NV - 1))
    c126 = jnp.full((16,), -126, jnp.int32)
    zf = zero
    nb = negbig
    vkvec = jnp.full((16,), VK, jnp.int32)

    def substep(t, renorm):
        toff = jnp.full((16,), t, dtype=jnp.int32) * vkvec
        news = []
        for k in range(NV):
            base = AB + 16 * k
            m_s = mbuf[pl.ds(base, 16)]
            m_1 = mbuf[pl.ds(base - 1, 16)]
            m_2 = mbuf[pl.ds(base - 2, 16)]
            p_s = pbuf[pl.ds(base, 16)]
            p_1 = pbuf[pl.ds(base - 1, 16)]
            p_2 = pbuf[pl.ds(base - 2, 16)]
            p_2 = jnp.where(sks[k], p_2, nb)
            g = plsc.load_gather(pv, [toff + eks[k]])
            pm = jnp.maximum(jnp.maximum(p_s, p_1), p_2)
            f_s = pow2(jnp.maximum(p_s - pm, c126))
            f_1 = pow2(jnp.maximum(p_1 - pm, c126))
            f_2 = pow2(jnp.maximum(p_2 - pm, c126))
            v = (m_s * f_s + m_1 * f_1 + m_2 * f_2) * g
            if k == NV - 1:
                v = jnp.where(padmask, v, zf)
            news.append((v, pm))
        for k in range(NV):
            v, pm = news[k]
            if renorm:
                eb = expfield(v)
                mbuf[pl.ds(AB + 16 * k, 16)] = v * pow2(127 - eb)
                pbuf[pl.ds(AB + 16 * k, 16)] = pm + (eb - 127)
            else:
                mbuf[pl.ds(AB + 16 * k, 16)] = v
                pbuf[pl.ds(AB + 16 * k, 16)] = pm

    def pair(j, carry):
        t = 1 + 2 * j
        substep(t, False)
        substep(t + 1, True)
        return carry

    hls = jnp.max(hlv[...])
    npairs = lax.shift_right_logical(hls - 1, 1)
    lax.fori_loop(0, npairs, pair, 0, unroll=False)
    last = 1 + 2 * npairs

    @pl.when(last < hls)
    def _():
        substep(last, True)

    pltpu.sync_copy(mbuf, mfin_hbm.at[b])
    pltpu.sync_copy(pbuf, pfin_hbm.at[b])


def _finish_kernel(m_ref, p_ref, sel_ref, sel2_ref, out_ref):
    m = m_ref[...]
    pe = p_ref[...].astype(jnp.float32)
    la = jnp.where(m > 0, jnp.log(m) + pe * LN2, NEGF)
    a_end = jnp.sum(jnp.where(sel_ref[...] > 0, la, 0.0),
                    axis=1, keepdims=True)
    a_end2 = jnp.sum(jnp.where(sel2_ref[...] > 0, la, 0.0),
                     axis=1, keepdims=True)
    nll = -jnp.logaddexp(a_end, a_end2)
    out_ref[...] = jnp.sum(nll, axis=(0, 1), keepdims=True) / B


@jax.jit
def kernel(hs_pad, hlens, ys_pad, ys_lens, W, b):
    ext = jnp.zeros((B, S), dtype=jnp.int32).at[:, 1::2].set(
        ys_pad.astype(jnp.int32))
    same = jnp.concatenate(
        [jnp.ones((B, 2), dtype=bool), ext[:, 2:] == ext[:, :-2]], axis=1)
    blank_pos = (jnp.arange(S) % 2 == 0)[None, :]
    skip_ok = jnp.logical_and(jnp.logical_not(blank_pos),
                              jnp.logical_not(same))
    ext_pad = jnp.zeros((B, SPAD), jnp.int32).at[:, :S].set(ext)
    skip_pad = jnp.zeros((B, SPAD), jnp.float32).at[:, :S].set(
        skip_ok.astype(jnp.float32))
    hl16 = jnp.broadcast_to(hlens.astype(jnp.int32)[:, None], (B, 16))

    s_last = 2 * ys_lens
    cols = jnp.arange(ABUF, dtype=jnp.int32)[None, :] - AB
    sel = (cols == s_last[:, None]).astype(jnp.float32)
    sel2 = (cols == jnp.maximum(s_last - 1, 0)[:, None]).astype(jnp.float32)
    b2 = b.reshape(1, V)

    probs = pl.pallas_call(
        _proj_kernel,
        grid=(NCHUNK,),
        in_specs=[
            pl.BlockSpec((B, TBP, D), lambda i: (0, i, 0)),
            pl.BlockSpec((D, V), lambda i: (0, 0)),
            pl.BlockSpec((1, V), lambda i: (0, 0)),
        ],
        out_specs=pl.BlockSpec((B, TBP, VK), lambda i: (0, i, 0)),
        out_shape=jax.ShapeDtypeStruct((B, T, VK), jnp.float32),
        compiler_params=pltpu.CompilerParams(
            dimension_semantics=("parallel",),
        ),
    )(hs_pad, W, b2)

    mesh = plsc.VectorSubcoreMesh(core_axis_name="c", subcore_axis_name="s")
    sc = pl.kernel(
        _sc_body,
        mesh=mesh,
        compiler_params=pltpu.CompilerParams(needs_layout_passes=False),
        out_type=[
            jax.ShapeDtypeStruct((B, ABUF), jnp.float32),
            jax.ShapeDtypeStruct((B, ABUF), jnp.int32),
        ],
        scratch_types=[
            pltpu.VMEM((T * VK,), jnp.float32),
            pltpu.VMEM((SPAD,), jnp.int32),
            pltpu.VMEM((SPAD,), jnp.float32),
            pltpu.VMEM((16,), jnp.int32),
            pltpu.VMEM((ABUF,), jnp.float32),
            pltpu.VMEM((ABUF,), jnp.int32),
        ],
    )
    mfin, pfin = sc(probs.reshape(B, T * VK), ext_pad, skip_pad, hl16)

    out = pl.pallas_call(
        _finish_kernel,
        in_specs=[
            pl.BlockSpec((B, ABUF), lambda: (0, 0)),
            pl.BlockSpec((B, ABUF), lambda: (0, 0)),
            pl.BlockSpec((B, ABUF), lambda: (0, 0)),
            pl.BlockSpec((B, ABUF), lambda: (0, 0)),
        ],
        out_specs=pl.BlockSpec((1, 1), lambda: (0, 0)),
        out_shape=jax.ShapeDtypeStruct((1, 1), jnp.float32),
    )(mfin, pfin, sel, sel2)
    return out[0, 0]
